# baseline (speedup 1.0000x reference)
import numpy as np
import jax
import jax.numpy as jnp
from jax import lax
from jax.experimental import pallas as pl
from jax.experimental.pallas import tpu as pltpu

_N_SAMPLES = 10
_ROTS = ((13, 15, 26, 6), (17, 29, 16, 24))
_VB = 1792


def _np_threefry2x32(k0, k1, x0, x1):
    mask = 0xFFFFFFFF
    ks = (k0, k1, (k0 ^ k1 ^ 0x1BD11BDA) & mask)
    x0 = (x0 + ks[0]) & mask
    x1 = (x1 + ks[1]) & mask
    for g in range(5):
        for r in _ROTS[g % 2]:
            x0 = (x0 + x1) & mask
            x1 = ((x1 << r) | (x1 >> (32 - r))) & mask
            x1 ^= x0
        x0 = (x0 + ks[(g + 1) % 3]) & mask
        x1 = (x1 + ks[(g + 2) % 3] + g + 1) & mask
    return x0, x1


_K0, _K1 = _np_threefry2x32(0, 0, 0, 1)
_K2 = (_K0 ^ _K1 ^ 0x1BD11BDA) & 0xFFFFFFFF
_KS = (_K0, _K1, _K2)


def _gumbel_bits(x1):
    x0 = jnp.full_like(x1, np.uint32(_K0))
    for g in range(5):
        for r in _ROTS[g % 2]:
            x0 = x0 + x1
            x1 = (x1 << np.uint32(r)) | (x1 >> np.uint32(32 - r))
            x1 = x0 ^ x1
        x0 = x0 + np.uint32(_KS[(g + 1) % 3])
        x1 = x1 + np.uint32((_KS[(g + 2) % 3] + g + 1) & 0xFFFFFFFF)
    return x0 ^ x1


def _gumbel_from_bits(bits):
    f = lax.bitcast_convert_type(
        (bits >> np.uint32(9)) | np.uint32(0x3F800000), jnp.float32)
    u = f - np.float32(1.0)
    return -jnp.log(-jnp.log(u))


def _scan_body(n_b, n_v, vb, nb, l_ref, bj_out, bz_out, se_out,
               m_ref, se_ref, bz_refs, bj_refs):
    j = pl.program_id(0)
    f32 = jnp.float32
    neg_inf = jnp.full((n_b, 128), -jnp.inf, f32)

    @pl.when(j == 0)
    def _init():
        m_ref[...] = neg_inf
        se_ref[...] = jnp.zeros((n_b, 128), f32)
        for s in range(_N_SAMPLES):
            bz_refs[s][...] = neg_inf
            bj_refs[s][...] = jnp.zeros((n_b, 128), jnp.int32)

    logits = l_ref[...]
    col = lax.broadcasted_iota(jnp.int32, (n_b, vb), 1) + j * vb
    valid = col < n_v
    lm = jnp.where(valid, logits, -jnp.inf)

    m_old = m_ref[...]
    m_new = jnp.maximum(m_old, jnp.max(lm, axis=1, keepdims=True))
    pse = jnp.sum(jnp.where(valid, jnp.exp(logits - m_new[:, 0:1]), 0.0),
                  axis=1, keepdims=True)
    se_ref[...] = se_ref[...] * jnp.exp(m_old - m_new) + pse
    m_ref[...] = m_new

    row = lax.broadcasted_iota(jnp.int32, (n_b, vb), 0)
    base = (row * n_v + col).astype(jnp.uint32)

    for s in range(_N_SAMPLES):
        bits = _gumbel_bits(base + np.uint32((s * n_b * n_v + _K1) & 0xFFFFFFFF))
        z = lm + _gumbel_from_bits(bits)
        zmax = jnp.max(z, axis=1, keepdims=True)
        bz_old = bz_refs[s][...]
        upd = zmax > bz_old
        bz_refs[s][...] = jnp.where(upd, zmax, bz_old)
        bj_refs[s][...] = jnp.where(upd, j, bj_refs[s][...])

    @pl.when(j == nb - 1)
    def _finish():
        for s in range(_N_SAMPLES):
            bj_out[s] = bj_refs[s][...]
            bz_out[s] = bz_refs[s][...]
        se_out[...] = se_ref[...]


def _resolve_body(n_b, n_v, vb, g_ref, bj_ref, bz_ref, se_ref, out_ref,
                  ids_ref):
    s = pl.program_id(0)
    f32 = jnp.float32
    logits = g_ref[0]
    blk = bj_ref[0][:, 0:1]
    col = blk * vb + lax.broadcasted_iota(jnp.int32, (n_b, vb), 1)
    lm = jnp.where(col < n_v, logits, -jnp.inf)
    row = lax.broadcasted_iota(jnp.int32, (n_b, vb), 0)
    base = (row * n_v + col).astype(jnp.uint32)
    s_off = (s * (n_b * n_v)).astype(jnp.uint32) + np.uint32(_K1)
    bits = _gumbel_bits(base + s_off)
    z = lm + _gumbel_from_bits(bits)
    idx = jnp.min(jnp.where(z == bz_ref[0][:, 0:1], col, n_v),
                  axis=1, keepdims=True)
    ids_ref[pl.ds(s, 1)] = jnp.broadcast_to(idx[None], (1, n_b, 128))

    @pl.when(s == _N_SAMPLES - 1)
    def _finish():
        ids = [ids_ref[t] for t in range(_N_SAMPLES)]
        agree = jnp.zeros((n_b, 128), f32)
        for t in range(_N_SAMPLES):
            agree += (ids[t] == ids[0]).astype(f32)
        agreement = agree * np.float32(1.0 / _N_SAMPLES)
        uniq = jnp.zeros((n_b, 128), f32)
        for t in range(_N_SAMPLES):
            seen = jnp.zeros((n_b, 128), jnp.bool_)
            for r in range(t):
                seen = jnp.logical_or(seen, ids[t] == ids[r])
            uniq += jnp.where(seen, 0.0, 1.0)
        unique_ratio = uniq * np.float32(1.0 / _N_SAMPLES)
        top_prob = np.float32(1.0) / se_ref[...]
        gap = agreement - top_prob
        lane = lax.broadcasted_iota(jnp.int32, (n_b, 3), 1)
        out_ref[...] = jnp.where(
            lane == 0, agreement[:, 0:1],
            jnp.where(lane == 1, unique_ratio[:, 0:1], gap[:, 0:1]))


def kernel(logits):
    n_b, n_v = logits.shape
    vb = _VB
    nb = pl.cdiv(n_v, vb)
    scratch = ([pltpu.VMEM((n_b, 128), jnp.float32)] * 2
               + [pltpu.VMEM((n_b, 128), jnp.float32)] * _N_SAMPLES
               + [pltpu.VMEM((n_b, 128), jnp.int32)] * _N_SAMPLES)

    def body1(l_ref, bj_out, bz_out, se_out, m_ref, se_ref, *rest):
        _scan_body(n_b, n_v, vb, nb, l_ref, bj_out, bz_out, se_out,
                   m_ref, se_ref, rest[:_N_SAMPLES], rest[_N_SAMPLES:])

    bj, bz, se = pl.pallas_call(
        body1,
        grid=(nb,),
        in_specs=[pl.BlockSpec((n_b, vb), lambda j: (0, j))],
        out_specs=[
            pl.BlockSpec((_N_SAMPLES, n_b, 128), lambda j: (0, 0, 0)),
            pl.BlockSpec((_N_SAMPLES, n_b, 128), lambda j: (0, 0, 0)),
            pl.BlockSpec((n_b, 128), lambda j: (0, 0)),
        ],
        out_shape=[
            jax.ShapeDtypeStruct((_N_SAMPLES, n_b, 128), jnp.int32),
            jax.ShapeDtypeStruct((_N_SAMPLES, n_b, 128), jnp.float32),
            jax.ShapeDtypeStruct((n_b, 128), jnp.float32),
        ],
        scratch_shapes=scratch,
        compiler_params=pltpu.CompilerParams(
            dimension_semantics=("arbitrary",)),
    )(logits)

    win = bj[:, :, 0]
    cols = win[:, :, None] * vb + jnp.arange(vb, dtype=jnp.int32)
    gath = jnp.take_along_axis(
        jnp.broadcast_to(logits[None], (_N_SAMPLES, n_b, n_v)),
        jnp.minimum(cols, n_v - 1), axis=2)

    def body2(g_ref, bj_ref, bz_ref, se_ref, out_ref, ids_ref):
        _resolve_body(n_b, n_v, vb, g_ref, bj_ref, bz_ref, se_ref, out_ref,
                      ids_ref)

    return pl.pallas_call(
        body2,
        grid=(_N_SAMPLES,),
        in_specs=[
            pl.BlockSpec((1, n_b, vb), lambda s: (s, 0, 0)),
            pl.BlockSpec((1, n_b, 128), lambda s: (s, 0, 0)),
            pl.BlockSpec((1, n_b, 128), lambda s: (s, 0, 0)),
            pl.BlockSpec((n_b, 128), lambda s: (0, 0)),
        ],
        out_specs=pl.BlockSpec((n_b, 3), lambda s: (0, 0)),
        out_shape=jax.ShapeDtypeStruct((n_b, 3), jnp.float32),
        scratch_shapes=[pltpu.VMEM((_N_SAMPLES, n_b, 128), jnp.int32)],
        compiler_params=pltpu.CompilerParams(
            dimension_semantics=("arbitrary",)),
    )(gath, bj, bz, se)


# speedup vs baseline: 1.0592x; 1.0592x over previous
import numpy as np
import jax
import jax.numpy as jnp
from jax import lax
from jax.experimental import pallas as pl
from jax.experimental.pallas import tpu as pltpu

_N_SAMPLES = 10
_ROTS = ((13, 15, 26, 6), (17, 29, 16, 24))
_TINY = np.float32(np.finfo(np.float32).tiny)


def _np_threefry2x32(k0, k1, x0, x1):
    mask = 0xFFFFFFFF
    ks = (k0, k1, (k0 ^ k1 ^ 0x1BD11BDA) & mask)
    x0 = (x0 + ks[0]) & mask
    x1 = (x1 + ks[1]) & mask
    for g in range(5):
        for r in _ROTS[g % 2]:
            x0 = (x0 + x1) & mask
            x1 = ((x1 << r) | (x1 >> (32 - r))) & mask
            x1 ^= x0
        x0 = (x0 + ks[(g + 1) % 3]) & mask
        x1 = (x1 + ks[(g + 2) % 3] + g + 1) & mask
    return x0, x1


_K0, _K1 = _np_threefry2x32(0, 0, 0, 1)
_K2 = (_K0 ^ _K1 ^ 0x1BD11BDA) & 0xFFFFFFFF
_KS = (_K0, _K1, _K2)


def _gumbel_bits(x1):
    x0 = jnp.full_like(x1, np.uint32(_K0))
    for g in range(5):
        for r in _ROTS[g % 2]:
            x0 = x0 + x1
            x1 = (x1 << np.uint32(r)) | (x1 >> np.uint32(32 - r))
            x1 = x0 ^ x1
        x0 = x0 + np.uint32(_KS[(g + 1) % 3])
        x1 = x1 + np.uint32((_KS[(g + 2) % 3] + g + 1) & 0xFFFFFFFF)
    return x0 ^ x1


def _sampler_body(n_b, n_v, vb, nb, l_ref, out_ref, m_ref, se_ref, bz_refs,
                  bi_refs):
    j = pl.program_id(0)
    f32 = jnp.float32
    neg_inf = jnp.full((n_b, 128), -jnp.inf, f32)

    @pl.when(j == 0)
    def _init():
        m_ref[...] = neg_inf
        se_ref[...] = jnp.zeros((n_b, 128), f32)
        for s in range(_N_SAMPLES):
            bz_refs[s][...] = neg_inf
            bi_refs[s][...] = jnp.zeros((n_b, 128), jnp.int32)

    logits = l_ref[...]
    col = lax.broadcasted_iota(jnp.int32, (n_b, vb), 1) + j * vb
    valid = col < n_v
    lm = jnp.where(valid, logits, -jnp.inf)

    m_old = m_ref[...]
    m_new = jnp.maximum(m_old, jnp.max(lm, axis=1, keepdims=True))
    pse = jnp.sum(jnp.where(valid, jnp.exp(logits - m_new[:, 0:1]), 0.0),
                  axis=1, keepdims=True)
    se_ref[...] = se_ref[...] * jnp.exp(m_old - m_new) + pse
    m_ref[...] = m_new

    row = lax.broadcasted_iota(jnp.int32, (n_b, vb), 0)
    base = (row * n_v + col).astype(jnp.uint32)

    for s in range(_N_SAMPLES):
        bits = _gumbel_bits(base + np.uint32((s * n_b * n_v + _K1) & 0xFFFFFFFF))
        f = lax.bitcast_convert_type(
            (bits >> np.uint32(9)) | np.uint32(0x3F800000), f32)
        u = f - np.float32(1.0)
        g = -jnp.log(-jnp.log(u))
        z = lm + g
        zmax = jnp.max(z, axis=1, keepdims=True)
        idx = jnp.min(jnp.where(z == zmax[:, 0:1], col, n_v),
                      axis=1, keepdims=True)
        bz_old = bz_refs[s][...]
        upd = zmax > bz_old
        bz_refs[s][...] = jnp.where(upd, zmax, bz_old)
        bi_refs[s][...] = jnp.where(upd, idx, bi_refs[s][...])

    @pl.when(j == nb - 1)
    def _finish():
        ids = [bi_refs[s][...] for s in range(_N_SAMPLES)]
        agree = jnp.zeros((n_b, 128), f32)
        for s in range(_N_SAMPLES):
            agree += (ids[s] == ids[0]).astype(f32)
        agreement = agree * np.float32(1.0 / _N_SAMPLES)
        uniq = jnp.zeros((n_b, 128), f32)
        for s in range(_N_SAMPLES):
            seen = jnp.zeros((n_b, 128), jnp.bool_)
            for t in range(s):
                seen = jnp.logical_or(seen, ids[s] == ids[t])
            uniq += jnp.where(seen, 0.0, 1.0)
        unique_ratio = uniq * np.float32(1.0 / _N_SAMPLES)
        top_prob = np.float32(1.0) / se_ref[...]
        gap = agreement - top_prob
        lane = lax.broadcasted_iota(jnp.int32, (n_b, 3), 1)
        out_ref[...] = jnp.where(
            lane == 0, agreement[:, 0:1],
            jnp.where(lane == 1, unique_ratio[:, 0:1], gap[:, 0:1]))


def kernel(logits):
    n_b, n_v = logits.shape
    vb = 1792
    nb = pl.cdiv(n_v, vb)
    scratch = ([pltpu.VMEM((n_b, 128), jnp.float32)] * 2
               + [pltpu.VMEM((n_b, 128), jnp.float32)] * _N_SAMPLES
               + [pltpu.VMEM((n_b, 128), jnp.int32)] * _N_SAMPLES)

    def body(l_ref, out_ref, m_ref, se_ref, *rest):
        bz_refs = rest[:_N_SAMPLES]
        bi_refs = rest[_N_SAMPLES:]
        _sampler_body(n_b, n_v, vb, nb, l_ref, out_ref, m_ref, se_ref,
                      bz_refs, bi_refs)

    return pl.pallas_call(
        body,
        grid=(nb,),
        in_specs=[pl.BlockSpec((n_b, vb), lambda j: (0, j))],
        out_specs=pl.BlockSpec((n_b, 3), lambda j: (0, 0)),
        out_shape=jax.ShapeDtypeStruct((n_b, 3), jnp.float32),
        scratch_shapes=scratch,
        compiler_params=pltpu.CompilerParams(
            dimension_semantics=("arbitrary",)),
    )(logits)


# speedup vs baseline: 1.0614x; 1.0021x over previous
import numpy as np
import jax
import jax.numpy as jnp
from jax import lax
from jax.experimental import pallas as pl
from jax.experimental.pallas import tpu as pltpu

_N_SAMPLES = 10
_ROTS = ((13, 15, 26, 6), (17, 29, 16, 24))
_TINY = np.float32(np.finfo(np.float32).tiny)


def _np_threefry2x32(k0, k1, x0, x1):
    mask = 0xFFFFFFFF
    ks = (k0, k1, (k0 ^ k1 ^ 0x1BD11BDA) & mask)
    x0 = (x0 + ks[0]) & mask
    x1 = (x1 + ks[1]) & mask
    for g in range(5):
        for r in _ROTS[g % 2]:
            x0 = (x0 + x1) & mask
            x1 = ((x1 << r) | (x1 >> (32 - r))) & mask
            x1 ^= x0
        x0 = (x0 + ks[(g + 1) % 3]) & mask
        x1 = (x1 + ks[(g + 2) % 3] + g + 1) & mask
    return x0, x1


_K0, _K1 = _np_threefry2x32(0, 0, 0, 1)
_K2 = (_K0 ^ _K1 ^ 0x1BD11BDA) & 0xFFFFFFFF
_KS = (_K0, _K1, _K2)


def _gumbel_bits(x1):
    x0 = jnp.full_like(x1, np.uint32(_K0))
    for g in range(5):
        for r in _ROTS[g % 2]:
            x0 = x0 + x1
            x1 = (x1 << np.uint32(r)) | (x1 >> np.uint32(32 - r))
            x1 = x0 ^ x1
        x0 = x0 + np.uint32(_KS[(g + 1) % 3])
        x1 = x1 + np.uint32((_KS[(g + 2) % 3] + g + 1) & 0xFFFFFFFF)
    return x0 ^ x1


def _sampler_body(n_b, n_v, vb, nb, l_ref, out_ref, m_ref, se_ref, bz_refs,
                  bi_refs):
    j = pl.program_id(0)
    f32 = jnp.float32
    neg_inf = jnp.full((n_b, 128), -jnp.inf, f32)

    @pl.when(j == 0)
    def _init():
        m_ref[...] = neg_inf
        se_ref[...] = jnp.zeros((n_b, 128), f32)
        for s in range(_N_SAMPLES):
            bz_refs[s][...] = neg_inf
            bi_refs[s][...] = jnp.zeros((n_b, 128), jnp.int32)

    logits = l_ref[...]
    col = lax.broadcasted_iota(jnp.int32, (n_b, vb), 1) + j * vb
    valid = col < n_v
    lm = jnp.where(valid, logits, -jnp.inf)

    m_old = m_ref[...]
    m_new = jnp.maximum(m_old, jnp.max(lm, axis=1, keepdims=True))
    pse = jnp.sum(jnp.where(valid, jnp.exp(logits - m_new[:, 0:1]), 0.0),
                  axis=1, keepdims=True)
    se_ref[...] = se_ref[...] * jnp.exp(m_old - m_new) + pse
    m_ref[...] = m_new

    row = lax.broadcasted_iota(jnp.int32, (n_b, vb), 0)
    base = (row * n_v + col).astype(jnp.uint32)

    for s in range(_N_SAMPLES):
        bits = _gumbel_bits(base + np.uint32((s * n_b * n_v + _K1) & 0xFFFFFFFF))
        f = lax.bitcast_convert_type(
            (bits >> np.uint32(9)) | np.uint32(0x3F800000), f32)
        u = f - np.float32(1.0)
        g = -jnp.log(-jnp.log(u))
        z = lm + g
        zmax = jnp.max(z, axis=1, keepdims=True)
        idx = jnp.min(jnp.where(z == zmax[:, 0:1], col, n_v),
                      axis=1, keepdims=True)
        bz_old = bz_refs[s][...]
        upd = zmax > bz_old
        bz_refs[s][...] = jnp.where(upd, zmax, bz_old)
        bi_refs[s][...] = jnp.where(upd, idx, bi_refs[s][...])

    @pl.when(j == nb - 1)
    def _finish():
        ids = [bi_refs[s][...] for s in range(_N_SAMPLES)]
        agree = jnp.zeros((n_b, 128), f32)
        for s in range(_N_SAMPLES):
            agree += (ids[s] == ids[0]).astype(f32)
        agreement = agree * np.float32(1.0 / _N_SAMPLES)
        uniq = jnp.zeros((n_b, 128), f32)
        for s in range(_N_SAMPLES):
            seen = jnp.zeros((n_b, 128), jnp.bool_)
            for t in range(s):
                seen = jnp.logical_or(seen, ids[s] == ids[t])
            uniq += jnp.where(seen, 0.0, 1.0)
        unique_ratio = uniq * np.float32(1.0 / _N_SAMPLES)
        top_prob = np.float32(1.0) / se_ref[...]
        gap = agreement - top_prob
        lane = lax.broadcasted_iota(jnp.int32, (n_b, 3), 1)
        out_ref[...] = jnp.where(
            lane == 0, agreement[:, 0:1],
            jnp.where(lane == 1, unique_ratio[:, 0:1], gap[:, 0:1]))


def kernel(logits):
    n_b, n_v = logits.shape
    vb = 1664
    nb = pl.cdiv(n_v, vb)
    scratch = ([pltpu.VMEM((n_b, 128), jnp.float32)] * 2
               + [pltpu.VMEM((n_b, 128), jnp.float32)] * _N_SAMPLES
               + [pltpu.VMEM((n_b, 128), jnp.int32)] * _N_SAMPLES)

    def body(l_ref, out_ref, m_ref, se_ref, *rest):
        bz_refs = rest[:_N_SAMPLES]
        bi_refs = rest[_N_SAMPLES:]
        _sampler_body(n_b, n_v, vb, nb, l_ref, out_ref, m_ref, se_ref,
                      bz_refs, bi_refs)

    return pl.pallas_call(
        body,
        grid=(nb,),
        in_specs=[pl.BlockSpec((n_b, vb), lambda j: (0, j))],
        out_specs=pl.BlockSpec((n_b, 3), lambda j: (0, 0)),
        out_shape=jax.ShapeDtypeStruct((n_b, 3), jnp.float32),
        scratch_shapes=scratch,
        compiler_params=pltpu.CompilerParams(
            dimension_semantics=("arbitrary",)),
    )(logits)
